# baseline (speedup 1.0000x reference)
import functools
import math

import jax
import jax.numpy as jnp
from jax import lax
from jax.experimental import pallas as pl
from jax.experimental.pallas import tpu as pltpu
from jax.experimental.pallas import tpu_sc as plsc

NC = 2
NS = 16
NW = NC * NS
CH = 128
NG = 4
IB = 32


def _sc_aggregate(xs, srcr, dstr, n, n_acc, k_ch):
    hd = xs.shape[2]
    nz = n_acc // NS

    mesh = plsc.VectorSubcoreMesh(core_axis_name="c", subcore_axis_name="s")

    @functools.partial(
        pl.kernel,
        out_type=(
            jax.ShapeDtypeStruct((NC, n_acc, hd), jnp.float32),
            jax.ShapeDtypeStruct((NC * n_acc,), jnp.float32),
        ),
        mesh=mesh,
        scratch_types=[
            pltpu.VMEM((IB, CH), jnp.int32),
            pltpu.VMEM((IB, CH), jnp.int32),
            pltpu.VMEM((2, NG, CH, hd), jnp.float32),
            pltpu.VMEM((CH,), jnp.float32),
            pltpu.VMEM((n_acc // NS,), jnp.float32),
            pltpu.VMEM_SHARED((n_acc, hd), jnp.float32),
            pltpu.VMEM_SHARED((n_acc,), jnp.float32),
        ] + [pltpu.SemaphoreType.DMA] * 6,
        compiler_params=pltpu.CompilerParams(use_tc_tiling_on_sc=False),
    )
    def sc_agg(xs_hbm, src_hbm, dst_hbm, ns_out, deg_out,
               srcv, dstv, rows, onesb, zb, acc, deg_sh, *sems):
        c = lax.axis_index("c")
        s = lax.axis_index("s")
        xh = xs_hbm.at[c]

        def zrow(i, carry):
            for cc in range(hd // 16):
                rows[0, 0, i, pl.ds(cc * 16, 16)] = jnp.zeros((16,),
                                                              jnp.float32)
            return carry
        lax.fori_loop(0, CH, zrow, 0)

        zbase = s * nz
        for kk in range(nz // CH):
            pltpu.sync_copy(rows.at[0, 0],
                            acc.at[pl.ds(zbase + kk * CH, CH)])
        zrem = nz % CH
        if zrem:
            pltpu.sync_copy(rows.at[0, 0, pl.ds(0, zrem)],
                            acc.at[pl.ds(zbase + (nz // CH) * CH, zrem)])

        for cc in range(CH // 16):
            onesb[pl.ds(cc * 16, 16)] = jnp.ones((16,), jnp.float32)
        def zdeg(i, carry):
            zb[pl.ds(i * 16, 16)] = jnp.zeros((16,), jnp.float32)
            return carry
        lax.fori_loop(0, nz // 16, zdeg, 0)
        pltpu.sync_copy(zb, deg_sh.at[pl.ds(s * nz, nz)])

        plsc.subcore_barrier()

        sem_g = sems[0:2]
        sem_s = sems[2:4]
        sem_d = sems[4:6]
        ngrp = IB // NG
        kmax = ngrp // 2 - 1

        def gather_fire(j, h, bb, sem):
            pltpu.async_copy(xh.at[dstv.at[j]], rows.at[h, bb], sem)

        def gather_wait(j, h, bb, sem):
            pltpu.make_async_copy(xh.at[dstv.at[j]], rows.at[h, bb],
                                  sem).wait()

        def scat_fire(j, h, bb, sem):
            pltpu.async_copy(rows.at[h, bb], acc.at[srcv.at[j]], sem,
                             add=True)

        def scat_wait(h, bb, sem):
            pltpu.make_async_copy(rows.at[h, bb], acc.at[srcv.at[0]],
                                  sem).wait()

        def dscat_fire(j, sem):
            pltpu.async_copy(onesb, deg_sh.at[srcv.at[j]], sem, add=True)

        def dscat_wait(sem):
            pltpu.make_async_copy(onesb, deg_sh.at[srcv.at[0]], sem).wait()

        for ib in range(k_ch // IB):
            cbase = s * k_ch + ib * IB
            pltpu.sync_copy(src_hbm.at[pl.ds(cbase, IB)], srcv)
            pltpu.sync_copy(dst_hbm.at[pl.ds(cbase, IB)], dstv)

            for bb in range(NG):
                gather_fire(bb, 0, bb, sem_g[0])

            def pair(k, carry):
                for h in (0, 1):
                    g = 2 * k + h
                    oh = 1 - h
                    jb = g * NG

                    def drain():
                        for bb in range(NG):
                            scat_wait(oh, bb, sem_s[oh])

                            @pl.when(c == (g + 1) % 2)
                            def _():
                                dscat_wait(sem_d[oh])
                    if h == 0:
                        pl.when(k > 0)(drain)
                    else:
                        drain()

                    def fire_next():
                        for bb in range(NG):
                            gather_fire(jb + NG + bb, oh, bb, sem_g[oh])
                    if h == 0:
                        fire_next()
                    else:
                        pl.when(k < kmax)(fire_next)

                    for bb in range(NG):
                        gather_wait(jb + bb, h, bb, sem_g[h])
                    for bb in range(NG):
                        scat_fire(jb + bb, h, bb, sem_s[h])

                        @pl.when(c == g % 2)
                        def _():
                            dscat_fire(jb + bb, sem_d[h])
                return carry
            lax.fori_loop(0, ngrp // 2, pair, 0)

            for bb in range(NG):
                scat_wait(1, bb, sem_s[1])

                @pl.when(c == (ngrp - 1) % 2)
                def _():
                    dscat_wait(sem_d[1])

        plsc.subcore_barrier()

        for kk in range(nz // CH):
            pltpu.sync_copy(acc.at[pl.ds(zbase + kk * CH, CH)],
                            ns_out.at[c, pl.ds(zbase + kk * CH, CH)])
        if zrem:
            ob = zbase + (nz // CH) * CH
            pltpu.sync_copy(acc.at[pl.ds(ob, zrem)],
                            ns_out.at[c, pl.ds(ob, zrem)])

        pltpu.sync_copy(deg_sh.at[pl.ds(s * nz, nz)],
                        deg_out.at[pl.ds(c * n_acc + s * nz, nz)])

    return sc_agg(xs, srcr, dstr)


def _tc_body(x_ref, ns_ref, deg_ref, dlt_ref, w_ref, bm_ref, bc_ref, gp_ref,
             out_ref):
    d = x_ref.shape[1]
    ns = jnp.concatenate([ns_ref[0], ns_ref[1]], axis=1)
    deg = jnp.clip(jnp.sum(deg_ref[...], axis=1), 1.0, None)
    mn = ns * (1.0 / deg)[:, None]
    xm = jnp.concatenate([x_ref[...], mn], axis=1)
    z = jnp.dot(xm, w_ref[...], preferred_element_type=jnp.float32)
    g = jax.nn.sigmoid(gp_ref[0] * dlt_ref[...][:, 0] + gp_ref[1])[:, None]
    h_mean = 0.5 * z[:, :d] + bm_ref[...]
    h_cat = z[:, d:] + bc_ref[...]
    out_ref[...] = h_mean + g * (h_cat - h_mean)


def kernel(x, edge_index, delta_agg, W_mean, b_mean, W_ego, b_ego, W_nb, b_nb,
           gate_weight, gate_bias):
    n, d = x.shape
    e = edge_index.shape[1]

    k_ch = IB * (-(-e // (NS * CH * IB)))
    e_pad = NS * k_ch * CH
    n_acc = (NS * 16) * (-(-(n + 1) // (NS * 16)))

    src = edge_index[0]
    dst = edge_index[1]
    pad = e_pad - e
    if pad:
        src = jnp.concatenate([src, jnp.full((pad,), n, jnp.int32)])
        dst = jnp.concatenate([dst, jnp.zeros((pad,), jnp.int32)])
    srcr = src.reshape(NS * k_ch, CH)
    dstr = dst.reshape(NS * k_ch, CH)
    hd = d // NC
    xs = jnp.stack([x[:, c * hd:(c + 1) * hd] for c in range(NC)])

    ns_p, deg_flat = _sc_aggregate(xs, srcr, dstr, n, n_acc, k_ch)
    deg_p = deg_flat.reshape(NC, n_acc).T

    top = jnp.concatenate(
        [W_mean.T, W_ego.T, jnp.zeros((d, d - W_ego.shape[0]), jnp.float32)],
        axis=1)
    bot = jnp.concatenate(
        [W_mean.T, jnp.zeros((d, W_ego.shape[0]), jnp.float32), W_nb.T],
        axis=1)
    wbig = jnp.concatenate([top, bot], axis=0)
    bm = b_mean[None, :]
    bc = jnp.concatenate([b_ego, b_nb])[None, :]
    gp = jnp.stack([gate_weight.astype(jnp.float32),
                    gate_bias.astype(jnp.float32)])
    dlt = delta_agg[:, None]

    r = 1000
    grid = (n // r,)
    h = pl.pallas_call(
        _tc_body,
        grid=grid,
        in_specs=[
            pl.BlockSpec((r, d), lambda i: (i, 0)),
            pl.BlockSpec((NC, r, d // NC), lambda i: (0, i, 0)),
            pl.BlockSpec((r, NC), lambda i: (i, 0)),
            pl.BlockSpec((r, 1), lambda i: (i, 0)),
            pl.BlockSpec((2 * d, 2 * d), lambda i: (0, 0)),
            pl.BlockSpec((1, d), lambda i: (0, 0)),
            pl.BlockSpec((1, d), lambda i: (0, 0)),
            pl.BlockSpec(memory_space=pltpu.SMEM),
        ],
        out_specs=pl.BlockSpec((r, d), lambda i: (i, 0)),
        out_shape=jax.ShapeDtypeStruct((n, d), jnp.float32),
    )(x, ns_p, deg_p, dlt, wbig, bm, bc, gp)
    return h


# speedup vs baseline: 1.5830x; 1.5830x over previous
import functools
import math

import jax
import jax.numpy as jnp
from jax import lax
from jax.experimental import pallas as pl
from jax.experimental.pallas import tpu as pltpu
from jax.experimental.pallas import tpu_sc as plsc

NC = 2
NS = 16
NW = NC * NS
CH = 128
NG = 2
IB = 32


def _sc_aggregate(xs, srcr, dstr, n, n_acc, k_ch):
    hd = xs.shape[2]
    nz = n_acc // NS

    mesh = plsc.VectorSubcoreMesh(core_axis_name="c", subcore_axis_name="s")

    @functools.partial(
        pl.kernel,
        out_type=(
            jax.ShapeDtypeStruct((NC, n_acc, hd), jnp.float32),
            jax.ShapeDtypeStruct((NC * n_acc,), jnp.float32),
        ),
        mesh=mesh,
        scratch_types=[
            pltpu.VMEM((IB, CH), jnp.int32),
            pltpu.VMEM((IB, CH), jnp.int32),
            pltpu.VMEM((2, NG, CH, hd), jnp.float32),
            pltpu.VMEM((CH,), jnp.float32),
            pltpu.VMEM((n_acc // NS,), jnp.float32),
            pltpu.VMEM_SHARED((n_acc, hd), jnp.float32),
            pltpu.VMEM_SHARED((n_acc, hd), jnp.float32),
            pltpu.VMEM_SHARED((n_acc,), jnp.float32),
        ] + [pltpu.SemaphoreType.DMA] * 6,
        compiler_params=pltpu.CompilerParams(use_tc_tiling_on_sc=False),
    )
    def sc_agg(xs_hbm, src_hbm, dst_hbm, ns_out, deg_out,
               srcv, dstv, rows, onesb, zb, acc, xsp, deg_sh, *sems):
        c = lax.axis_index("c")
        s = lax.axis_index("s")
        xh = xs_hbm.at[c]

        def zrow(i, carry):
            for cc in range(hd // 16):
                rows[0, 0, i, pl.ds(cc * 16, 16)] = jnp.zeros((16,),
                                                              jnp.float32)
            return carry
        lax.fori_loop(0, CH, zrow, 0)

        zbase = s * nz
        for kk in range(nz // CH):
            pltpu.sync_copy(rows.at[0, 0],
                            acc.at[pl.ds(zbase + kk * CH, CH)])
        zrem = nz % CH
        if zrem:
            pltpu.sync_copy(rows.at[0, 0, pl.ds(0, zrem)],
                            acc.at[pl.ds(zbase + (nz // CH) * CH, zrem)])

        for cc in range(CH // 16):
            onesb[pl.ds(cc * 16, 16)] = jnp.ones((16,), jnp.float32)
        def zdeg(i, carry):
            zb[pl.ds(i * 16, 16)] = jnp.zeros((16,), jnp.float32)
            return carry
        lax.fori_loop(0, nz // 16, zdeg, 0)
        pltpu.sync_copy(zb, deg_sh.at[pl.ds(s * nz, nz)])

        pltpu.sync_copy(xh.at[pl.ds(s * nz, nz)], xsp.at[pl.ds(s * nz, nz)])

        plsc.subcore_barrier()

        sem_g = sems[0:2]
        sem_s = sems[2:4]
        sem_d = sems[4:6]
        ngrp = IB // NG
        kmax = ngrp // 2 - 1

        def gather_fire(j, h, bb, sem):
            pltpu.async_copy(xsp.at[dstv.at[j]], rows.at[h, bb], sem)

        def gather_wait(j, h, bb, sem):
            pltpu.make_async_copy(xsp.at[dstv.at[j]], rows.at[h, bb],
                                  sem).wait()

        def scat_fire(j, h, bb, sem):
            pltpu.async_copy(rows.at[h, bb], acc.at[srcv.at[j]], sem,
                             add=True)

        def scat_wait(h, bb, sem):
            pltpu.make_async_copy(rows.at[h, bb], acc.at[srcv.at[0]],
                                  sem).wait()

        def dscat_fire(j, sem):
            pltpu.async_copy(onesb, deg_sh.at[srcv.at[j]], sem, add=True)

        def dscat_wait(sem):
            pltpu.make_async_copy(onesb, deg_sh.at[srcv.at[0]], sem).wait()

        for ib in range(k_ch // IB):
            cbase = s * k_ch + ib * IB
            pltpu.sync_copy(src_hbm.at[pl.ds(cbase, IB)], srcv)
            pltpu.sync_copy(dst_hbm.at[pl.ds(cbase, IB)], dstv)

            for bb in range(NG):
                gather_fire(bb, 0, bb, sem_g[0])

            def pair(k, carry):
                for h in (0, 1):
                    g = 2 * k + h
                    oh = 1 - h
                    jb = g * NG

                    def drain():
                        for bb in range(NG):
                            scat_wait(oh, bb, sem_s[oh])

                            @pl.when(c == (g + 1) % 2)
                            def _():
                                dscat_wait(sem_d[oh])
                    if h == 0:
                        pl.when(k > 0)(drain)
                    else:
                        drain()

                    def fire_next():
                        for bb in range(NG):
                            gather_fire(jb + NG + bb, oh, bb, sem_g[oh])
                    if h == 0:
                        fire_next()
                    else:
                        pl.when(k < kmax)(fire_next)

                    for bb in range(NG):
                        gather_wait(jb + bb, h, bb, sem_g[h])
                    for bb in range(NG):
                        scat_fire(jb + bb, h, bb, sem_s[h])

                        @pl.when(c == g % 2)
                        def _():
                            dscat_fire(jb + bb, sem_d[h])
                return carry
            lax.fori_loop(0, ngrp // 2, pair, 0)

            for bb in range(NG):
                scat_wait(1, bb, sem_s[1])

                @pl.when(c == (ngrp - 1) % 2)
                def _():
                    dscat_wait(sem_d[1])

        plsc.subcore_barrier()

        for kk in range(nz // CH):
            pltpu.sync_copy(acc.at[pl.ds(zbase + kk * CH, CH)],
                            ns_out.at[c, pl.ds(zbase + kk * CH, CH)])
        if zrem:
            ob = zbase + (nz // CH) * CH
            pltpu.sync_copy(acc.at[pl.ds(ob, zrem)],
                            ns_out.at[c, pl.ds(ob, zrem)])

        pltpu.sync_copy(deg_sh.at[pl.ds(s * nz, nz)],
                        deg_out.at[pl.ds(c * n_acc + s * nz, nz)])

    return sc_agg(xs, srcr, dstr)


def _tc_body(x_ref, ns_ref, deg_ref, dlt_ref, w_ref, bm_ref, bc_ref, gp_ref,
             out_ref):
    d = x_ref.shape[1]
    ns = jnp.concatenate([ns_ref[0], ns_ref[1]], axis=1)
    deg = jnp.clip(jnp.sum(deg_ref[...], axis=1), 1.0, None)
    mn = ns * (1.0 / deg)[:, None]
    xm = jnp.concatenate([x_ref[...], mn], axis=1)
    z = jnp.dot(xm, w_ref[...], preferred_element_type=jnp.float32)
    g = jax.nn.sigmoid(gp_ref[0] * dlt_ref[...][:, 0] + gp_ref[1])[:, None]
    h_mean = 0.5 * z[:, :d] + bm_ref[...]
    h_cat = z[:, d:] + bc_ref[...]
    out_ref[...] = h_mean + g * (h_cat - h_mean)


def kernel(x, edge_index, delta_agg, W_mean, b_mean, W_ego, b_ego, W_nb, b_nb,
           gate_weight, gate_bias):
    n, d = x.shape
    e = edge_index.shape[1]

    k_ch = IB * (-(-e // (NS * CH * IB)))
    e_pad = NS * k_ch * CH
    n_acc = (NS * 16) * (-(-(n + 1) // (NS * 16)))

    src = edge_index[0]
    dst = edge_index[1]
    pad = e_pad - e
    if pad:
        src = jnp.concatenate([src, jnp.full((pad,), n, jnp.int32)])
        dst = jnp.concatenate([dst, jnp.zeros((pad,), jnp.int32)])
    srcr = src.reshape(NS * k_ch, CH)
    dstr = dst.reshape(NS * k_ch, CH)
    hd = d // NC
    xp = jnp.concatenate(
        [x, jnp.zeros((n_acc - n, d), jnp.float32)], axis=0)
    xs = jnp.stack([xp[:, c * hd:(c + 1) * hd] for c in range(NC)])

    ns_p, deg_flat = _sc_aggregate(xs, srcr, dstr, n, n_acc, k_ch)
    deg_p = deg_flat.reshape(NC, n_acc).T

    top = jnp.concatenate(
        [W_mean.T, W_ego.T, jnp.zeros((d, d - W_ego.shape[0]), jnp.float32)],
        axis=1)
    bot = jnp.concatenate(
        [W_mean.T, jnp.zeros((d, W_ego.shape[0]), jnp.float32), W_nb.T],
        axis=1)
    wbig = jnp.concatenate([top, bot], axis=0)
    bm = b_mean[None, :]
    bc = jnp.concatenate([b_ego, b_nb])[None, :]
    gp = jnp.stack([gate_weight.astype(jnp.float32),
                    gate_bias.astype(jnp.float32)])
    dlt = delta_agg[:, None]

    r = 1000
    grid = (n // r,)
    h = pl.pallas_call(
        _tc_body,
        grid=grid,
        in_specs=[
            pl.BlockSpec((r, d), lambda i: (i, 0)),
            pl.BlockSpec((NC, r, d // NC), lambda i: (0, i, 0)),
            pl.BlockSpec((r, NC), lambda i: (i, 0)),
            pl.BlockSpec((r, 1), lambda i: (i, 0)),
            pl.BlockSpec((2 * d, 2 * d), lambda i: (0, 0)),
            pl.BlockSpec((1, d), lambda i: (0, 0)),
            pl.BlockSpec((1, d), lambda i: (0, 0)),
            pl.BlockSpec(memory_space=pltpu.SMEM),
        ],
        out_specs=pl.BlockSpec((r, d), lambda i: (i, 0)),
        out_shape=jax.ShapeDtypeStruct((n, d), jnp.float32),
    )(x, ns_p, deg_p, dlt, wbig, bm, bc, gp)
    return h


# speedup vs baseline: 1.7567x; 1.1097x over previous
import functools

import jax
import jax.numpy as jnp
from jax import lax
from jax.experimental import pallas as pl
from jax.experimental.pallas import tpu as pltpu
from jax.experimental.pallas import tpu_sc as plsc

NC = 2
NS = 16
CH = 128
NG = 2
IB = 32


def _sc_aggregate(x, ei, n, n_acc):
    d = x.shape[1]
    hd = d // NC
    e = ei.shape[1]
    nz = n_acc // NS
    chunks = e // CH
    kc = chunks // NS
    extras = chunks % NS
    nfb = kc // IB
    rem = kc % IB
    assert rem % (2 * NG) == 0, "partial block must pipeline evenly"
    blk_sizes = [IB] * nfb + ([rem] if rem else [])
    full_tiles = n // nz
    tail_rows = n - full_tiles * nz

    mesh = plsc.VectorSubcoreMesh(core_axis_name="c", subcore_axis_name="s")

    @functools.partial(
        pl.kernel,
        out_type=(
            jax.ShapeDtypeStruct((NC, n_acc, hd), jnp.float32),
            jax.ShapeDtypeStruct((NC * n_acc,), jnp.float32),
        ),
        mesh=mesh,
        scratch_types=[
            pltpu.VMEM((IB * CH,), jnp.int32),
            pltpu.VMEM((IB * CH,), jnp.int32),
            pltpu.VMEM((2, NG, CH, hd), jnp.float32),
            pltpu.VMEM((CH,), jnp.float32),
            pltpu.VMEM((n_acc // NS,), jnp.float32),
            pltpu.VMEM_SHARED((n_acc, hd), jnp.float32),
            pltpu.VMEM_SHARED((n_acc, hd), jnp.float32),
            pltpu.VMEM_SHARED((n_acc,), jnp.float32),
        ] + [pltpu.SemaphoreType.DMA] * 6,
        compiler_params=pltpu.CompilerParams(use_tc_tiling_on_sc=False),
    )
    def sc_agg(x_hbm, ei_hbm, ns_out, deg_out,
               srcv, dstv, rows, onesb, zb, acc, xsp, deg_sh, *sems):
        c = lax.axis_index("c")
        s = lax.axis_index("s")

        def zrow(i, carry):
            for cc in range(hd // 16):
                rows[0, 0, i, pl.ds(cc * 16, 16)] = jnp.zeros((16,),
                                                              jnp.float32)
            return carry
        lax.fori_loop(0, CH, zrow, 0)

        zbase = s * nz
        for kk in range(nz // CH):
            pltpu.sync_copy(rows.at[0, 0],
                            acc.at[pl.ds(zbase + kk * CH, CH)])
        zrem = nz % CH
        if zrem:
            pltpu.sync_copy(rows.at[0, 0, pl.ds(0, zrem)],
                            acc.at[pl.ds(zbase + (nz // CH) * CH, zrem)])

        for cc in range(CH // 16):
            onesb[pl.ds(cc * 16, 16)] = jnp.ones((16,), jnp.float32)

        def zdeg(i, carry):
            zb[pl.ds(i * 16, 16)] = jnp.zeros((16,), jnp.float32)
            return carry
        lax.fori_loop(0, nz // 16, zdeg, 0)
        pltpu.sync_copy(zb, deg_sh.at[pl.ds(s * nz, nz)])

        @pl.when(s < full_tiles)
        def _():
            pltpu.sync_copy(x_hbm.at[pl.ds(zbase, nz), pl.ds(c * hd, hd)],
                            xsp.at[pl.ds(zbase, nz)])
        if tail_rows:
            @pl.when(s == full_tiles)
            def _():
                pltpu.sync_copy(
                    x_hbm.at[pl.ds(zbase, tail_rows), pl.ds(c * hd, hd)],
                    xsp.at[pl.ds(zbase, tail_rows)])

        plsc.subcore_barrier()

        sem_g = sems[0:2]
        sem_s = sems[2:4]
        sem_d = sems[4:6]

        def idxs(ref, j):
            off = pl.multiple_of(j * CH, CH)
            return ref.at[pl.ds(off, CH)]

        def gather_fire(j, h, bb, sem):
            pltpu.async_copy(xsp.at[idxs(dstv, j)], rows.at[h, bb], sem)

        def gather_wait(j, h, bb, sem):
            pltpu.make_async_copy(xsp.at[idxs(dstv, j)], rows.at[h, bb],
                                  sem).wait()

        def scat_fire(j, h, bb, sem):
            pltpu.async_copy(rows.at[h, bb], acc.at[idxs(srcv, j)], sem,
                             add=True)

        def scat_wait(h, bb, sem):
            pltpu.make_async_copy(rows.at[h, bb], acc.at[idxs(srcv, 0)],
                                  sem).wait()

        def dscat_fire(j, sem):
            pltpu.async_copy(onesb, deg_sh.at[idxs(srcv, j)], sem, add=True)

        def dscat_wait(sem):
            pltpu.make_async_copy(onesb, deg_sh.at[idxs(srcv, 0)],
                                  sem).wait()

        cdone = 0
        for bsz in blk_sizes:
            ngrp = bsz // NG
            kmax = ngrp // 2 - 1
            cbase = (s * kc + cdone) * CH
            pltpu.sync_copy(ei_hbm.at[0, pl.ds(cbase, bsz * CH)],
                            srcv.at[pl.ds(0, bsz * CH)])
            pltpu.sync_copy(ei_hbm.at[1, pl.ds(cbase, bsz * CH)],
                            dstv.at[pl.ds(0, bsz * CH)])

            for bb in range(NG):
                gather_fire(bb, 0, bb, sem_g[0])

            def pair(k, carry):
                for h in (0, 1):
                    g = 2 * k + h
                    oh = 1 - h
                    jb = g * NG

                    def drain():
                        for bb in range(NG):
                            scat_wait(oh, bb, sem_s[oh])

                            @pl.when(c == (g + 1) % 2)
                            def _():
                                dscat_wait(sem_d[oh])
                    if h == 0:
                        pl.when(k > 0)(drain)
                    else:
                        drain()

                    def fire_next():
                        for bb in range(NG):
                            gather_fire(jb + NG + bb, oh, bb, sem_g[oh])
                    if h == 0:
                        fire_next()
                    else:
                        pl.when(k < kmax)(fire_next)

                    for bb in range(NG):
                        gather_wait(jb + bb, h, bb, sem_g[h])
                    for bb in range(NG):
                        scat_fire(jb + bb, h, bb, sem_s[h])

                        @pl.when(c == g % 2)
                        def _():
                            dscat_fire(jb + bb, sem_d[h])
                return carry
            lax.fori_loop(0, ngrp // 2, pair, 0)

            for bb in range(NG):
                scat_wait(1, bb, sem_s[1])

                @pl.when(c == (ngrp - 1) % 2)
                def _():
                    dscat_wait(sem_d[1])

            cdone += bsz

        if extras:
            @pl.when(s < extras)
            def _():
                eoff = pl.multiple_of(kc * NS * CH + s * CH, CH)
                pltpu.sync_copy(ei_hbm.at[0, pl.ds(eoff, CH)],
                                srcv.at[pl.ds(0, CH)])
                pltpu.sync_copy(ei_hbm.at[1, pl.ds(eoff, CH)],
                                dstv.at[pl.ds(0, CH)])
                gather_fire(0, 0, 0, sem_g[0])
                gather_wait(0, 0, 0, sem_g[0])
                pltpu.sync_copy(rows.at[0, 0], acc.at[idxs(srcv, 0)],
                                add=True)

                @pl.when(c == s % 2)
                def _():
                    pltpu.sync_copy(onesb, deg_sh.at[idxs(srcv, 0)],
                                    add=True)

        plsc.subcore_barrier()

        for kk in range(nz // CH):
            pltpu.sync_copy(acc.at[pl.ds(zbase + kk * CH, CH)],
                            ns_out.at[c, pl.ds(zbase + kk * CH, CH)])
        if zrem:
            ob = zbase + (nz // CH) * CH
            pltpu.sync_copy(acc.at[pl.ds(ob, zrem)],
                            ns_out.at[c, pl.ds(ob, zrem)])

        pltpu.sync_copy(deg_sh.at[pl.ds(s * nz, nz)],
                        deg_out.at[pl.ds(c * n_acc + s * nz, nz)])

    return sc_agg(x, ei)


def _tc_body(x_ref, ns_ref, dg0_ref, dg1_ref, dlt_ref, w_ref, bm_ref, bc_ref,
             gp_ref, out_ref):
    d = x_ref.shape[1]
    ns = jnp.concatenate([ns_ref[0], ns_ref[1]], axis=1)
    deg = jnp.clip(dg0_ref[...][:, 0] + dg1_ref[...][:, 0], 1.0, None)
    mn = ns * (1.0 / deg)[:, None]
    xm = jnp.concatenate([x_ref[...], mn], axis=1)
    z = jnp.dot(xm, w_ref[...], preferred_element_type=jnp.float32)
    g = jax.nn.sigmoid(gp_ref[0] * dlt_ref[...][:, 0] + gp_ref[1])[:, None]
    h_mean = 0.5 * z[:, :d] + bm_ref[...]
    h_cat = z[:, d:] + bc_ref[...]
    out_ref[...] = h_mean + g * (h_cat - h_mean)


def kernel(x, edge_index, delta_agg, W_mean, b_mean, W_ego, b_ego, W_nb, b_nb,
           gate_weight, gate_bias):
    n, d = x.shape
    e = edge_index.shape[1]

    n_acc = (NS * 16) * (-(-(n + 1) // (NS * 16)))

    if e % CH:
        padn = CH - e % CH
        epad = jnp.concatenate(
            [jnp.full((1, padn), n, jnp.int32),
             jnp.zeros((1, padn), jnp.int32)], axis=0)
        edge_index = jnp.concatenate([edge_index, epad], axis=1)

    ns_p, deg_flat = _sc_aggregate(x, edge_index, n, n_acc)
    dg0 = deg_flat[:n_acc, None]
    dg1 = deg_flat[n_acc:, None]

    hd2 = W_ego.shape[0]
    top = jnp.concatenate(
        [W_mean.T, W_ego.T, jnp.zeros((d, d - hd2), jnp.float32)], axis=1)
    bot = jnp.concatenate(
        [W_mean.T, jnp.zeros((d, hd2), jnp.float32), W_nb.T], axis=1)
    wbig = jnp.concatenate([top, bot], axis=0)
    bm = b_mean[None, :]
    bc = jnp.concatenate([b_ego, b_nb])[None, :]
    gp = jnp.stack([gate_weight.astype(jnp.float32),
                    gate_bias.astype(jnp.float32)])
    dlt = delta_agg[:, None]

    r = 1000
    grid = (n // r,)
    h = pl.pallas_call(
        _tc_body,
        grid=grid,
        in_specs=[
            pl.BlockSpec((r, d), lambda i: (i, 0)),
            pl.BlockSpec((NC, r, d // NC), lambda i: (0, i, 0)),
            pl.BlockSpec((r, 1), lambda i: (i, 0)),
            pl.BlockSpec((r, 1), lambda i: (i, 0)),
            pl.BlockSpec((r, 1), lambda i: (i, 0)),
            pl.BlockSpec((2 * d, 2 * d), lambda i: (0, 0)),
            pl.BlockSpec((1, d), lambda i: (0, 0)),
            pl.BlockSpec((1, d), lambda i: (0, 0)),
            pl.BlockSpec(memory_space=pltpu.SMEM),
        ],
        out_specs=pl.BlockSpec((r, d), lambda i: (i, 0)),
        out_shape=jax.ShapeDtypeStruct((n, d), jnp.float32),
    )(x, ns_p, dg0, dg1, dlt, wbig, bm, bc, gp)
    return h


# speedup vs baseline: 1.8076x; 1.0290x over previous
import functools

import jax
import jax.numpy as jnp
from jax import lax
from jax.experimental import pallas as pl
from jax.experimental.pallas import tpu as pltpu
from jax.experimental.pallas import tpu_sc as plsc

NC = 2
NS = 16
CH = 128
NG = 2
IB = 32


def _sc_aggregate(x, ei, n, n_acc):
    d = x.shape[1]
    hd = d // NC
    e = ei.shape[1]
    nz = n_acc // NS
    chunks = e // CH
    kc = chunks // NS
    extras = chunks % NS
    nfb = kc // IB
    rem = kc % IB
    assert rem % (2 * NG) == 0, "partial block must pipeline evenly"
    blk_sizes = [IB] * nfb + ([rem] if rem else [])
    full_tiles = n // nz
    tail_rows = n - full_tiles * nz

    mesh = plsc.VectorSubcoreMesh(core_axis_name="c", subcore_axis_name="s")

    @functools.partial(
        pl.kernel,
        out_type=(
            jax.ShapeDtypeStruct((NC, n_acc, hd), jnp.float32),
            jax.ShapeDtypeStruct((NC * n_acc,), jnp.float32),
        ),
        mesh=mesh,
        scratch_types=[
            pltpu.VMEM((IB * CH,), jnp.int32),
            pltpu.VMEM((IB * CH,), jnp.int32),
            pltpu.VMEM((2, NG, CH, hd), jnp.float32),
            pltpu.VMEM((CH,), jnp.float32),
            pltpu.VMEM((n_acc // NS,), jnp.float32),
            pltpu.VMEM_SHARED((n_acc, hd), jnp.float32),
            pltpu.VMEM_SHARED((n_acc, hd), jnp.float32),
            pltpu.VMEM_SHARED((n_acc,), jnp.float32),
        ] + [pltpu.SemaphoreType.DMA] * 6,
        compiler_params=pltpu.CompilerParams(use_tc_tiling_on_sc=False),
    )
    def sc_agg(x_hbm, ei_hbm, ns_out, deg_out,
               srcv, dstv, rows, onesb, zb, acc, xsp, deg_sh, *sems):
        c = lax.axis_index("c")
        s = lax.axis_index("s")

        def zrow(i, carry):
            for cc in range(hd // 16):
                rows[0, 0, i, pl.ds(cc * 16, 16)] = jnp.zeros((16,),
                                                              jnp.float32)
            return carry
        lax.fori_loop(0, CH, zrow, 0)

        zbase = s * nz
        for kk in range(nz // CH):
            pltpu.sync_copy(rows.at[0, 0],
                            acc.at[pl.ds(zbase + kk * CH, CH)])
        zrem = nz % CH
        if zrem:
            pltpu.sync_copy(rows.at[0, 0, pl.ds(0, zrem)],
                            acc.at[pl.ds(zbase + (nz // CH) * CH, zrem)])

        for cc in range(CH // 16):
            onesb[pl.ds(cc * 16, 16)] = jnp.ones((16,), jnp.float32)

        def zdeg(i, carry):
            zb[pl.ds(i * 16, 16)] = jnp.zeros((16,), jnp.float32)
            return carry
        lax.fori_loop(0, nz // 16, zdeg, 0)
        pltpu.sync_copy(zb, deg_sh.at[pl.ds(s * nz, nz)])

        @pl.when(s < full_tiles)
        def _():
            pltpu.sync_copy(x_hbm.at[pl.ds(zbase, nz), pl.ds(c * hd, hd)],
                            xsp.at[pl.ds(zbase, nz)])
        if tail_rows:
            @pl.when(s == full_tiles)
            def _():
                pltpu.sync_copy(
                    x_hbm.at[pl.ds(zbase, tail_rows), pl.ds(c * hd, hd)],
                    xsp.at[pl.ds(zbase, tail_rows)])

        plsc.subcore_barrier()

        sem_g = sems[0:2]
        sem_s = sems[2:4]
        sem_d = sems[4:6]

        def idxs(ref, j):
            off = pl.multiple_of(j * CH, CH)
            return ref.at[pl.ds(off, CH)]

        def gather_fire(j, h, bb, sem):
            pltpu.async_copy(xsp.at[idxs(dstv, j)], rows.at[h, bb], sem)

        def gather_wait(j, h, bb, sem):
            pltpu.make_async_copy(xsp.at[idxs(dstv, j)], rows.at[h, bb],
                                  sem).wait()

        def scat_fire(j, h, bb, sem):
            pltpu.async_copy(rows.at[h, bb], acc.at[idxs(srcv, j)], sem,
                             add=True)

        def scat_wait(h, bb, sem):
            pltpu.make_async_copy(rows.at[h, bb], acc.at[idxs(srcv, 0)],
                                  sem).wait()

        def dscat_fire(j, sem):
            pltpu.async_copy(onesb, deg_sh.at[idxs(srcv, j)], sem, add=True)

        def dscat_wait(sem):
            pltpu.make_async_copy(onesb, deg_sh.at[idxs(srcv, 0)],
                                  sem).wait()

        cdone = 0
        for bsz in blk_sizes:
            ngrp = bsz // NG
            kmax = ngrp // 2 - 1
            cbase = (s * kc + cdone) * CH
            pltpu.sync_copy(ei_hbm.at[0, pl.ds(cbase, bsz * CH)],
                            srcv.at[pl.ds(0, bsz * CH)])
            pltpu.sync_copy(ei_hbm.at[1, pl.ds(cbase, bsz * CH)],
                            dstv.at[pl.ds(0, bsz * CH)])

            for bb in range(NG):
                gather_fire(bb, 0, bb, sem_g[0])

            def pair(k, carry):
                for h in (0, 1):
                    g = 2 * k + h
                    oh = 1 - h
                    jb = g * NG

                    def drain():
                        for bb in range(NG):
                            scat_wait(oh, bb, sem_s[oh])

                            @pl.when(c == (g + 1) % 2)
                            def _():
                                dscat_wait(sem_d[oh])
                    if h == 0:
                        pl.when(k > 0)(drain)
                    else:
                        drain()

                    def fire_next():
                        for bb in range(NG):
                            gather_fire(jb + NG + bb, oh, bb, sem_g[oh])
                    if h == 0:
                        fire_next()
                    else:
                        pl.when(k < kmax)(fire_next)

                    for bb in range(NG):
                        gather_wait(jb + bb, h, bb, sem_g[h])
                    for bb in range(NG):
                        scat_fire(jb + bb, h, bb, sem_s[h])

                        @pl.when(c == g % 2)
                        def _():
                            dscat_fire(jb + bb, sem_d[h])
                return carry
            lax.fori_loop(0, ngrp // 2, pair, 0)

            for bb in range(NG):
                scat_wait(1, bb, sem_s[1])

                @pl.when(c == (ngrp - 1) % 2)
                def _():
                    dscat_wait(sem_d[1])

            cdone += bsz

        if extras:
            @pl.when(s < extras)
            def _():
                eoff = pl.multiple_of(kc * NS * CH + s * CH, CH)
                pltpu.sync_copy(ei_hbm.at[0, pl.ds(eoff, CH)],
                                srcv.at[pl.ds(0, CH)])
                pltpu.sync_copy(ei_hbm.at[1, pl.ds(eoff, CH)],
                                dstv.at[pl.ds(0, CH)])
                gather_fire(0, 0, 0, sem_g[0])
                gather_wait(0, 0, 0, sem_g[0])
                pltpu.sync_copy(rows.at[0, 0], acc.at[idxs(srcv, 0)],
                                add=True)

                @pl.when(c == s % 2)
                def _():
                    pltpu.sync_copy(onesb, deg_sh.at[idxs(srcv, 0)],
                                    add=True)

        plsc.subcore_barrier()

        for kk in range(nz // CH):
            pltpu.sync_copy(acc.at[pl.ds(zbase + kk * CH, CH)],
                            ns_out.at[c, pl.ds(zbase + kk * CH, CH)])
        if zrem:
            ob = zbase + (nz // CH) * CH
            pltpu.sync_copy(acc.at[pl.ds(ob, zrem)],
                            ns_out.at[c, pl.ds(ob, zrem)])

        pltpu.sync_copy(deg_sh.at[pl.ds(s * nz, nz)],
                        deg_out.at[pl.ds(c * n_acc + s * nz, nz)])

    return sc_agg(x, ei)


def _tc_body(x_ref, ns_ref, vec_ref, w_ref, bm_ref, bc_ref,
             gp_ref, out_ref):
    d = x_ref.shape[1]
    vec = vec_ref[...]
    ns = jnp.concatenate([ns_ref[0], ns_ref[1]], axis=1)
    deg = jnp.clip(vec[:, 1:2] + vec[:, 2:3], 1.0, None)
    mn = ns * (1.0 / deg)
    xm = jnp.concatenate([x_ref[...], mn], axis=1)
    z = jnp.dot(xm, w_ref[...], preferred_element_type=jnp.float32)
    g = jax.nn.sigmoid(gp_ref[0] * vec[:, 0:1] + gp_ref[1])
    h_mean = 0.5 * z[:, :d] + bm_ref[...]
    h_cat = z[:, d:] + bc_ref[...]
    out_ref[...] = h_mean + g * (h_cat - h_mean)


def kernel(x, edge_index, delta_agg, W_mean, b_mean, W_ego, b_ego, W_nb, b_nb,
           gate_weight, gate_bias):
    n, d = x.shape
    e = edge_index.shape[1]

    n_acc = (NS * 16) * (-(-(n + 1) // (NS * 16)))

    if e % CH:
        padn = CH - e % CH
        epad = jnp.concatenate(
            [jnp.full((1, padn), n, jnp.int32),
             jnp.zeros((1, padn), jnp.int32)], axis=0)
        edge_index = jnp.concatenate([edge_index, epad], axis=1)

    ns_p, deg_flat = _sc_aggregate(x, edge_index, n, n_acc)

    hd2 = W_ego.shape[0]
    top = jnp.concatenate(
        [W_mean.T, W_ego.T, jnp.zeros((d, d - hd2), jnp.float32)], axis=1)
    bot = jnp.concatenate(
        [W_mean.T, jnp.zeros((d, hd2), jnp.float32), W_nb.T], axis=1)
    wbig = jnp.concatenate([top, bot], axis=0)
    bm = b_mean[None, :]
    bc = jnp.concatenate([b_ego, b_nb])[None, :]
    gp = jnp.stack([gate_weight.astype(jnp.float32),
                    gate_bias.astype(jnp.float32)])
    dlt_pad = jnp.concatenate(
        [delta_agg, jnp.zeros((n_acc - n,), jnp.float32)])
    vec = jnp.stack(
        [dlt_pad, deg_flat[:n_acc], deg_flat[n_acc:]], axis=1)

    r = 2048
    rr = r // 128
    grid = (-(-n // r),)
    h = pl.pallas_call(
        _tc_body,
        grid=grid,
        in_specs=[
            pl.BlockSpec((r, d), lambda i: (i, 0)),
            pl.BlockSpec((NC, r, d // NC), lambda i: (0, i, 0)),
            pl.BlockSpec((r, 3), lambda i: (i, 0)),
            pl.BlockSpec((2 * d, 2 * d), lambda i: (0, 0)),
            pl.BlockSpec((1, d), lambda i: (0, 0)),
            pl.BlockSpec((1, d), lambda i: (0, 0)),
            pl.BlockSpec(memory_space=pltpu.SMEM),
        ],
        out_specs=pl.BlockSpec((r, d), lambda i: (i, 0)),
        out_shape=jax.ShapeDtypeStruct((n, d), jnp.float32),
    )(x, ns_p, vec, wbig, bm, bc, gp)
    return h


# speedup vs baseline: 1.8630x; 1.0306x over previous
import functools

import jax
import jax.numpy as jnp
from jax import lax
from jax.experimental import pallas as pl
from jax.experimental.pallas import tpu as pltpu
from jax.experimental.pallas import tpu_sc as plsc

NC = 2
NS = 16
CH = 128
NG = 2
IB = 32


def _sc_aggregate(x, ei, n, n_acc):
    d = x.shape[1]
    hd = d // NC
    e = ei.shape[1]
    nz = n_acc // NS
    chunks = e // CH
    kc = chunks // NS
    extras = chunks % NS
    nfb = kc // IB
    rem = kc % IB
    assert rem % (2 * NG) == 0, "partial block must pipeline evenly"
    blk_sizes = [IB] * nfb + ([rem] if rem else [])
    blk_hbm = [i % 2 == 1 for i in range(len(blk_sizes))]
    full_tiles = n // nz
    tail_rows = n - full_tiles * nz

    mesh = plsc.VectorSubcoreMesh(core_axis_name="c", subcore_axis_name="s")

    @functools.partial(
        pl.kernel,
        out_type=(
            jax.ShapeDtypeStruct((NC, n_acc, hd), jnp.float32),
            jax.ShapeDtypeStruct((NC * n_acc,), jnp.float32),
        ),
        mesh=mesh,
        scratch_types=[
            pltpu.VMEM((IB * CH,), jnp.int32),
            pltpu.VMEM((IB * CH,), jnp.int32),
            pltpu.VMEM((2, NG, CH, hd), jnp.float32),
            pltpu.VMEM((CH,), jnp.float32),
            pltpu.VMEM((n_acc // NS,), jnp.float32),
            pltpu.VMEM_SHARED((n_acc, hd), jnp.float32),
            pltpu.VMEM_SHARED((n_acc, hd), jnp.float32),
            pltpu.VMEM_SHARED((n_acc,), jnp.float32),
        ] + [pltpu.SemaphoreType.DMA] * 6,
        compiler_params=pltpu.CompilerParams(use_tc_tiling_on_sc=False),
    )
    def sc_agg(x_hbm, xs_hbm, ei_hbm, ns_out, deg_out,
               srcv, dstv, rows, onesb, zb, acc, xsp, deg_sh, *sems):
        c = lax.axis_index("c")
        s = lax.axis_index("s")
        xh = xs_hbm.at[c]

        def zrow(i, carry):
            for cc in range(hd // 16):
                rows[0, 0, i, pl.ds(cc * 16, 16)] = jnp.zeros((16,),
                                                              jnp.float32)
            return carry
        lax.fori_loop(0, CH, zrow, 0)

        zbase = s * nz
        for kk in range(nz // CH):
            pltpu.sync_copy(rows.at[0, 0],
                            acc.at[pl.ds(zbase + kk * CH, CH)])
        zrem = nz % CH
        if zrem:
            pltpu.sync_copy(rows.at[0, 0, pl.ds(0, zrem)],
                            acc.at[pl.ds(zbase + (nz // CH) * CH, zrem)])

        for cc in range(CH // 16):
            onesb[pl.ds(cc * 16, 16)] = jnp.ones((16,), jnp.float32)

        def zdeg(i, carry):
            zb[pl.ds(i * 16, 16)] = jnp.zeros((16,), jnp.float32)
            return carry
        lax.fori_loop(0, nz // 16, zdeg, 0)
        pltpu.sync_copy(zb, deg_sh.at[pl.ds(s * nz, nz)])

        @pl.when(s < full_tiles)
        def _():
            pltpu.sync_copy(x_hbm.at[pl.ds(zbase, nz), pl.ds(c * hd, hd)],
                            xsp.at[pl.ds(zbase, nz)])
        if tail_rows:
            @pl.when(s == full_tiles)
            def _():
                pltpu.sync_copy(
                    x_hbm.at[pl.ds(zbase, tail_rows), pl.ds(c * hd, hd)],
                    xsp.at[pl.ds(zbase, tail_rows)])

        plsc.subcore_barrier()

        sem_g = sems[0:2]
        sem_s = sems[2:4]
        sem_d = sems[4:6]

        def idxs(ref, j):
            off = pl.multiple_of(j * CH, CH)
            return ref.at[pl.ds(off, CH)]

        def gather_fire(j, h, bb, sem, hbm=False):
            srcr = xh if hbm else xsp
            pltpu.async_copy(srcr.at[idxs(dstv, j)], rows.at[h, bb], sem)

        def gather_wait(j, h, bb, sem, hbm=False):
            srcr = xh if hbm else xsp
            pltpu.make_async_copy(srcr.at[idxs(dstv, j)], rows.at[h, bb],
                                  sem).wait()

        def scat_fire(j, h, bb, sem):
            pltpu.async_copy(rows.at[h, bb], acc.at[idxs(srcv, j)], sem,
                             add=True)

        def scat_wait(h, bb, sem):
            pltpu.make_async_copy(rows.at[h, bb], acc.at[idxs(srcv, 0)],
                                  sem).wait()

        def dscat_fire(j, sem):
            pltpu.async_copy(onesb, deg_sh.at[idxs(srcv, j)], sem, add=True)

        def dscat_wait(sem):
            pltpu.make_async_copy(onesb, deg_sh.at[idxs(srcv, 0)],
                                  sem).wait()

        cdone = 0
        for bsz, use_hbm in zip(blk_sizes, blk_hbm):
            ngrp = bsz // NG
            kmax = ngrp // 2 - 1
            cbase = (s * kc + cdone) * CH
            pltpu.sync_copy(ei_hbm.at[0, pl.ds(cbase, bsz * CH)],
                            srcv.at[pl.ds(0, bsz * CH)])
            pltpu.sync_copy(ei_hbm.at[1, pl.ds(cbase, bsz * CH)],
                            dstv.at[pl.ds(0, bsz * CH)])

            for bb in range(NG):
                gather_fire(bb, 0, bb, sem_g[0], use_hbm)

            def pair(k, carry):
                for h in (0, 1):
                    g = 2 * k + h
                    oh = 1 - h
                    jb = g * NG

                    def drain():
                        for bb in range(NG):
                            scat_wait(oh, bb, sem_s[oh])

                            @pl.when(c == (g + 1) % 2)
                            def _():
                                dscat_wait(sem_d[oh])
                    if h == 0:
                        pl.when(k > 0)(drain)
                    else:
                        drain()

                    def fire_next():
                        for bb in range(NG):
                            gather_fire(jb + NG + bb, oh, bb, sem_g[oh],
                                        use_hbm)
                    if h == 0:
                        fire_next()
                    else:
                        pl.when(k < kmax)(fire_next)

                    for bb in range(NG):
                        gather_wait(jb + bb, h, bb, sem_g[h], use_hbm)
                    for bb in range(NG):
                        scat_fire(jb + bb, h, bb, sem_s[h])

                        @pl.when(c == g % 2)
                        def _():
                            dscat_fire(jb + bb, sem_d[h])
                return carry
            lax.fori_loop(0, ngrp // 2, pair, 0)

            for bb in range(NG):
                scat_wait(1, bb, sem_s[1])

                @pl.when(c == (ngrp - 1) % 2)
                def _():
                    dscat_wait(sem_d[1])

            cdone += bsz

        if extras:
            @pl.when(s < extras)
            def _():
                eoff = pl.multiple_of(kc * NS * CH + s * CH, CH)
                pltpu.sync_copy(ei_hbm.at[0, pl.ds(eoff, CH)],
                                srcv.at[pl.ds(0, CH)])
                pltpu.sync_copy(ei_hbm.at[1, pl.ds(eoff, CH)],
                                dstv.at[pl.ds(0, CH)])
                gather_fire(0, 0, 0, sem_g[0])
                gather_wait(0, 0, 0, sem_g[0])
                pltpu.sync_copy(rows.at[0, 0], acc.at[idxs(srcv, 0)],
                                add=True)

                @pl.when(c == s % 2)
                def _():
                    pltpu.sync_copy(onesb, deg_sh.at[idxs(srcv, 0)],
                                    add=True)

        plsc.subcore_barrier()

        for kk in range(nz // CH):
            pltpu.sync_copy(acc.at[pl.ds(zbase + kk * CH, CH)],
                            ns_out.at[c, pl.ds(zbase + kk * CH, CH)])
        if zrem:
            ob = zbase + (nz // CH) * CH
            pltpu.sync_copy(acc.at[pl.ds(ob, zrem)],
                            ns_out.at[c, pl.ds(ob, zrem)])

        pltpu.sync_copy(deg_sh.at[pl.ds(s * nz, nz)],
                        deg_out.at[pl.ds(c * n_acc + s * nz, nz)])

    hd_ = d // NC
    xs = jnp.stack([x[:, cc * hd_:(cc + 1) * hd_] for cc in range(NC)])
    return sc_agg(x, xs, ei)


def _tc_body(x_ref, ns_ref, vec_ref, w_ref, bm_ref, bc_ref,
             gp_ref, out_ref):
    d = x_ref.shape[1]
    vec = vec_ref[...]
    ns = jnp.concatenate([ns_ref[0], ns_ref[1]], axis=1)
    deg = jnp.clip(vec[:, 1:2] + vec[:, 2:3], 1.0, None)
    mn = ns * (1.0 / deg)
    xm = jnp.concatenate([x_ref[...], mn], axis=1)
    z = jnp.dot(xm, w_ref[...], preferred_element_type=jnp.float32)
    g = jax.nn.sigmoid(gp_ref[0] * vec[:, 0:1] + gp_ref[1])
    h_mean = 0.5 * z[:, :d] + bm_ref[...]
    h_cat = z[:, d:] + bc_ref[...]
    out_ref[...] = h_mean + g * (h_cat - h_mean)


def kernel(x, edge_index, delta_agg, W_mean, b_mean, W_ego, b_ego, W_nb, b_nb,
           gate_weight, gate_bias):
    n, d = x.shape
    e = edge_index.shape[1]

    n_acc = (NS * 16) * (-(-(n + 1) // (NS * 16)))

    if e % CH:
        padn = CH - e % CH
        epad = jnp.concatenate(
            [jnp.full((1, padn), n, jnp.int32),
             jnp.zeros((1, padn), jnp.int32)], axis=0)
        edge_index = jnp.concatenate([edge_index, epad], axis=1)

    ns_p, deg_flat = _sc_aggregate(x, edge_index, n, n_acc)

    hd2 = W_ego.shape[0]
    top = jnp.concatenate(
        [W_mean.T, W_ego.T, jnp.zeros((d, d - hd2), jnp.float32)], axis=1)
    bot = jnp.concatenate(
        [W_mean.T, jnp.zeros((d, hd2), jnp.float32), W_nb.T], axis=1)
    wbig = jnp.concatenate([top, bot], axis=0)
    bm = b_mean[None, :]
    bc = jnp.concatenate([b_ego, b_nb])[None, :]
    gp = jnp.stack([gate_weight.astype(jnp.float32),
                    gate_bias.astype(jnp.float32)])
    dlt_pad = jnp.concatenate(
        [delta_agg, jnp.zeros((n_acc - n,), jnp.float32)])
    vec = jnp.stack(
        [dlt_pad, deg_flat[:n_acc], deg_flat[n_acc:]], axis=1)

    r = 2048
    rr = r // 128
    grid = (-(-n // r),)
    h = pl.pallas_call(
        _tc_body,
        grid=grid,
        in_specs=[
            pl.BlockSpec((r, d), lambda i: (i, 0)),
            pl.BlockSpec((NC, r, d // NC), lambda i: (0, i, 0)),
            pl.BlockSpec((r, 3), lambda i: (i, 0)),
            pl.BlockSpec((2 * d, 2 * d), lambda i: (0, 0)),
            pl.BlockSpec((1, d), lambda i: (0, 0)),
            pl.BlockSpec((1, d), lambda i: (0, 0)),
            pl.BlockSpec(memory_space=pltpu.SMEM),
        ],
        out_specs=pl.BlockSpec((r, d), lambda i: (i, 0)),
        out_shape=jax.ShapeDtypeStruct((n, d), jnp.float32),
    )(x, ns_p, vec, wbig, bm, bc, gp)
    return h
